# baseline (speedup 1.0000x reference)
import functools

import jax
import jax.numpy as jnp
from jax import lax
from jax.experimental import pallas as pl
from jax.experimental.pallas import tpu as pltpu
from jax.experimental.pallas import tpu_sc as plsc

H = 256
IN_DIM = 768
N_NODE = 10000
E_EDGE = 160000
EL_EDGE = 160000

NCORE = 2
NSUB = 16
NWORK = NCORE * NSUB
SEG_CHUNK = 40
SEG_PER_W = E_EDGE // NWORK
SEG_ITERS = SEG_PER_W // SEG_CHUNK
GAT_CHUNK = 40
GAT_PER_W = EL_EDGE // NWORK
GAT_ITERS = GAT_PER_W // GAT_CHUNK
ZROWS = 624

_SC_MESH = dict(core_axis_name="c", subcore_axis_name="s")


def _gelu(x):
    return 0.5 * x * (1.0 + lax.erf(x * (2.0 ** -0.5)))


def _zero_stripe(zeros_hbm, out_ref, w):
    pltpu.sync_copy(zeros_hbm.at[pl.ds(w * ZROWS, ZROWS)],
                    out_ref.at[pl.ds(w * ZROWS, ZROWS)])

    @pl.when(w == NSUB - 1)
    def _tail():
        pltpu.sync_copy(zeros_hbm.at[pl.ds(NSUB * ZROWS, N_NODE - NSUB * ZROWS)],
                        out_ref.at[pl.ds(NSUB * ZROWS, N_NODE - NSUB * ZROWS)])


def _segsum_sc(table, src, dst, zeros_h):
    @functools.partial(
        pl.kernel,
        out_type=(jax.ShapeDtypeStruct((N_NODE, H), jnp.float32),
                  jax.ShapeDtypeStruct((N_NODE, H), jnp.float32)),
        mesh=plsc.VectorSubcoreMesh(**_SC_MESH),
        scratch_types=[
            pltpu.VMEM((SEG_CHUNK,), jnp.int32),
            pltpu.VMEM((SEG_CHUNK,), jnp.int32),
            pltpu.VMEM((SEG_CHUNK, H), jnp.float32),
            pltpu.SemaphoreType.DMA,
        ],
    )
    def k(table_h, src_h, dst_h, zeros_hbm, out_a, out_b, src_v, dst_v, rows_v, sem):
        c = lax.axis_index("c")
        w = lax.axis_index("s")
        wid = w * NCORE + c

        @pl.when(c == 0)
        def _za():
            _zero_stripe(zeros_hbm, out_a, w)

        @pl.when(c == 1)
        def _zb():
            _zero_stripe(zeros_hbm, out_b, w)

        plsc.subcore_barrier()
        base0 = wid * SEG_PER_W

        def step(i, carry):
            base = base0 + i * SEG_CHUNK
            pltpu.sync_copy(src_h.at[pl.ds(base, SEG_CHUNK)], src_v)
            pltpu.sync_copy(dst_h.at[pl.ds(base, SEG_CHUNK)], dst_v)
            pltpu.async_copy(table_h.at[src_v], rows_v, sem).wait()

            @pl.when(c == 0)
            def _sa():
                pltpu.sync_copy(rows_v, out_a.at[dst_v], add=True)

            @pl.when(c == 1)
            def _sb():
                pltpu.sync_copy(rows_v, out_b.at[dst_v], add=True)

            return carry

        lax.fori_loop(0, SEG_ITERS, step, 0)

    return k(table, src, dst, zeros_h)


def _counts_sc(dst_a, dst_b, ones_h, zeros_h):
    @functools.partial(
        pl.kernel,
        out_type=tuple(jax.ShapeDtypeStruct((N_NODE, H), jnp.float32)
                       for _ in range(4)),
        mesh=plsc.VectorSubcoreMesh(**_SC_MESH),
        scratch_types=[
            pltpu.VMEM((SEG_CHUNK,), jnp.int32),
            pltpu.VMEM((SEG_CHUNK, H), jnp.float32),
        ],
    )
    def k(dst_a_h, dst_b_h, ones_hbm, zeros_hbm,
          out_a0, out_a1, out_b0, out_b1, dst_v, ones_v):
        c = lax.axis_index("c")
        w = lax.axis_index("s")
        wid = w * NCORE + c
        pltpu.sync_copy(ones_hbm, ones_v)

        @pl.when(c == 0)
        def _za():
            _zero_stripe(zeros_hbm, out_a0, w)
            _zero_stripe(zeros_hbm, out_b0, w)

        @pl.when(c == 1)
        def _zb():
            _zero_stripe(zeros_hbm, out_a1, w)
            _zero_stripe(zeros_hbm, out_b1, w)

        plsc.subcore_barrier()
        base0 = wid * SEG_PER_W

        def step_a(i, carry):
            pltpu.sync_copy(dst_a_h.at[pl.ds(base0 + i * SEG_CHUNK, SEG_CHUNK)], dst_v)

            @pl.when(c == 0)
            def _sa():
                pltpu.sync_copy(ones_v, out_a0.at[dst_v], add=True)

            @pl.when(c == 1)
            def _sb():
                pltpu.sync_copy(ones_v, out_a1.at[dst_v], add=True)

            return carry

        def step_b(i, carry):
            pltpu.sync_copy(dst_b_h.at[pl.ds(base0 + i * SEG_CHUNK, SEG_CHUNK)], dst_v)

            @pl.when(c == 0)
            def _sa():
                pltpu.sync_copy(ones_v, out_b0.at[dst_v], add=True)

            @pl.when(c == 1)
            def _sb():
                pltpu.sync_copy(ones_v, out_b1.at[dst_v], add=True)

            return carry

        lax.fori_loop(0, SEG_ITERS, step_a, 0)
        lax.fori_loop(0, SEG_ITERS, step_b, 0)

    return k(dst_a, dst_b, ones_h, zeros_h)


def _gather2_sc(t0, t1, i0, i1):
    @functools.partial(
        pl.kernel,
        out_type=(jax.ShapeDtypeStruct((EL_EDGE, H), jnp.float32),
                  jax.ShapeDtypeStruct((EL_EDGE, H), jnp.float32)),
        mesh=plsc.VectorSubcoreMesh(**_SC_MESH),
        scratch_types=[
            pltpu.VMEM((GAT_CHUNK,), jnp.int32),
            pltpu.VMEM((GAT_CHUNK,), jnp.int32),
            pltpu.VMEM((GAT_CHUNK, H), jnp.float32),
            pltpu.VMEM((GAT_CHUNK, H), jnp.float32),
            pltpu.SemaphoreType.DMA,
            pltpu.SemaphoreType.DMA,
        ],
    )
    def k(t0_h, t1_h, i0_h, i1_h, g0_h, g1_h, i0_v, i1_v, r0_v, r1_v, s0, s1):
        c = lax.axis_index("c")
        w = lax.axis_index("s")
        wid = w * NCORE + c
        base0 = wid * GAT_PER_W

        def step(i, carry):
            base = base0 + i * GAT_CHUNK
            pltpu.sync_copy(i0_h.at[pl.ds(base, GAT_CHUNK)], i0_v)
            pltpu.sync_copy(i1_h.at[pl.ds(base, GAT_CHUNK)], i1_v)
            cp0 = pltpu.async_copy(t0_h.at[i0_v], r0_v, s0)
            cp1 = pltpu.async_copy(t1_h.at[i1_v], r1_v, s1)
            cp0.wait()
            cp1.wait()
            pltpu.sync_copy(r0_v, g0_h.at[pl.ds(base, GAT_CHUNK)])
            pltpu.sync_copy(r1_v, g1_h.at[pl.ds(base, GAT_CHUNK)])
            return carry

        lax.fori_loop(0, GAT_ITERS, step, 0)

    return k(t0, t1, i0, i1)


def _proj_tc(x, W, b, bm=1000):
    M, K = x.shape
    N = W.shape[1]

    def body(x_ref, w_ref, b_ref, o_ref):
        acc = jnp.dot(x_ref[...], w_ref[...], preferred_element_type=jnp.float32)
        o_ref[...] = _gelu(acc + b_ref[...])

    return pl.pallas_call(
        body,
        grid=(M // bm,),
        in_specs=[pl.BlockSpec((bm, K), lambda i: (i, 0)),
                  pl.BlockSpec((K, N), lambda i: (0, 0)),
                  pl.BlockSpec((1, N), lambda i: (0, 0))],
        out_specs=pl.BlockSpec((bm, N), lambda i: (i, 0)),
        out_shape=jax.ShapeDtypeStruct((M, N), jnp.float32),
    )(x, W, b.reshape(1, N))


def _combine_tc(agg_a, agg_b, cnt_a, cnt_b, z_dst, Wl, Wr, b, bm=1000):
    M = agg_a.shape[0]

    def body(aa_ref, ab_ref, ca_ref, cb_ref, z_ref, wl_ref, wr_ref, b_ref, o_ref):
        cnt = ca_ref[...][:, 0:1] + cb_ref[...][:, 0:1]
        mean = (aa_ref[...] + ab_ref[...]) / jnp.maximum(cnt, 1.0)
        acc = jnp.dot(mean, wl_ref[...], preferred_element_type=jnp.float32)
        acc += jnp.dot(z_ref[...], wr_ref[...], preferred_element_type=jnp.float32)
        o_ref[...] = _gelu(acc + b_ref[...])

    return pl.pallas_call(
        body,
        grid=(M // bm,),
        in_specs=[pl.BlockSpec((bm, H), lambda i: (i, 0)),
                  pl.BlockSpec((bm, H), lambda i: (i, 0)),
                  pl.BlockSpec((bm, H), lambda i: (i, 0)),
                  pl.BlockSpec((bm, H), lambda i: (i, 0)),
                  pl.BlockSpec((bm, H), lambda i: (i, 0)),
                  pl.BlockSpec((H, H), lambda i: (0, 0)),
                  pl.BlockSpec((H, H), lambda i: (0, 0)),
                  pl.BlockSpec((1, H), lambda i: (0, 0))],
        out_specs=pl.BlockSpec((bm, H), lambda i: (i, 0)),
        out_shape=jax.ShapeDtypeStruct((M, H), jnp.float32),
    )(agg_a, agg_b, cnt_a, cnt_b, z_dst, Wl, Wr, b.reshape(1, H))


def _edge_head_tc(zt_g, zs_g, Wa, Wb, Wc, b1, W2, b2, bm=1000):
    M = zt_g.shape[0]
    NO = W2.shape[1]

    def body(zt_ref, zs_ref, wa_ref, wb_ref, wc_ref, b1_ref, w2_ref, b2_ref, o_ref):
        zt = zt_ref[...]
        zs = zs_ref[...]
        d = jnp.abs(zt - zs)
        acc = jnp.dot(zt, wa_ref[...], preferred_element_type=jnp.float32)
        acc += jnp.dot(zs, wb_ref[...], preferred_element_type=jnp.float32)
        acc += jnp.dot(d, wc_ref[...], preferred_element_type=jnp.float32)
        g = _gelu(acc + b1_ref[...])
        o_ref[...] = jnp.dot(g, w2_ref[...], preferred_element_type=jnp.float32) + b2_ref[...]

    return pl.pallas_call(
        body,
        grid=(M // bm,),
        in_specs=[pl.BlockSpec((bm, H), lambda i: (i, 0)),
                  pl.BlockSpec((bm, H), lambda i: (i, 0)),
                  pl.BlockSpec((H, H), lambda i: (0, 0)),
                  pl.BlockSpec((H, H), lambda i: (0, 0)),
                  pl.BlockSpec((H, H), lambda i: (0, 0)),
                  pl.BlockSpec((1, H), lambda i: (0, 0)),
                  pl.BlockSpec((H, NO), lambda i: (0, 0)),
                  pl.BlockSpec((1, NO), lambda i: (0, 0))],
        out_specs=pl.BlockSpec((bm, NO), lambda i: (i, 0)),
        out_shape=jax.ShapeDtypeStruct((M, NO), jnp.float32),
    )(zt_g, zs_g, Wa, Wb, Wc, b1.reshape(1, H), W2, b2.reshape(1, NO))


def _node_heads_tc(z, W1, b1, W2, b2, bm=1000):
    M = z.shape[0]

    def body(z_ref, w1_ref, b1_ref, w2_ref, b2_ref, o_ref):
        h = _gelu(jnp.dot(z_ref[...], w1_ref[...], preferred_element_type=jnp.float32)
                  + b1_ref[...])
        o = jnp.dot(h, w2_ref[...], preferred_element_type=jnp.float32) + b2_ref[...]
        col = lax.broadcasted_iota(jnp.int32, o.shape, 1)
        o_ref[...] = jnp.where(col == 1, jax.nn.sigmoid(o), o)

    return pl.pallas_call(
        body,
        grid=(M // bm,),
        in_specs=[pl.BlockSpec((bm, H), lambda i: (i, 0)),
                  pl.BlockSpec((H, H), lambda i: (0, 0)),
                  pl.BlockSpec((1, H), lambda i: (0, 0)),
                  pl.BlockSpec((H, 2), lambda i: (0, 0)),
                  pl.BlockSpec((1, 2), lambda i: (0, 0))],
        out_specs=pl.BlockSpec((bm, 2), lambda i: (i, 0)),
        out_shape=jax.ShapeDtypeStruct((M, 2), jnp.float32),
    )(z, W1, b1.reshape(1, H), W2, b2.reshape(1, 2))


def kernel(x_transcript, x_symptom, edge_index_ts, edge_index_st, edge_label_index,
           Wt_proj, bt_proj, Ws_proj, bs_proj,
           Wl0_ts, bl0_ts, Wr0_ts, Wl0_st, bl0_st, Wr0_st,
           Wl1_ts, bl1_ts, Wr1_ts, Wl1_st, bl1_st, Wr1_st,
           We1, be1, We2, be2, Wb1, bb1, Wb2, bb2, Wsc1, bsc1, Wsc2, bsc2):
    ei_ts = edge_index_ts.astype(jnp.int32)
    ei_st = edge_index_st.astype(jnp.int32)
    eli = edge_label_index.astype(jnp.int32)
    src_ts, dst_ts = ei_ts[0], ei_ts[1]
    src_st, dst_st = ei_st[0], ei_st[1]

    zeros_big = jnp.zeros((N_NODE, H), jnp.float32)
    zeros_cnt = zeros_big
    ones_cnt = jnp.ones((SEG_CHUNK, H), jnp.float32)

    z_t = _proj_tc(x_transcript, Wt_proj, bt_proj)
    z_s = _proj_tc(x_symptom, Ws_proj, bs_proj)

    cts0, cts1, cst0, cst1 = _counts_sc(dst_ts, dst_st, ones_cnt, zeros_cnt)

    layers = ((Wl0_ts, bl0_ts, Wr0_ts, Wl0_st, bl0_st, Wr0_st),
              (Wl1_ts, bl1_ts, Wr1_ts, Wl1_st, bl1_st, Wr1_st))
    for (Wl_ts, bl_ts, Wr_ts, Wl_st, bl_st, Wr_st) in layers:
        agg_s0, agg_s1 = _segsum_sc(z_t, src_ts, dst_ts, zeros_big)
        agg_t0, agg_t1 = _segsum_sc(z_s, src_st, dst_st, zeros_big)
        new_s = _combine_tc(agg_s0, agg_s1, cts0, cts1, z_s, Wl_ts, Wr_ts, bl_ts)
        new_t = _combine_tc(agg_t0, agg_t1, cst0, cst1, z_t, Wl_st, Wr_st, bl_st)
        z_t, z_s = new_t, new_s

    zt_g, zs_g = _gather2_sc(z_t, z_s, eli[0], eli[1])

    Wa, Wb, Wc = We1[:H], We1[H:2 * H], We1[2 * H:]
    edge_logits = _edge_head_tc(zt_g, zs_g, Wa, Wb, Wc, be1, We2, be2)

    W1 = jnp.concatenate([Wb1, Wsc1], axis=1)
    b1 = jnp.concatenate([bb1, bsc1])
    W2 = jnp.zeros((H, 2), jnp.float32)
    W2 = W2.at[:H // 2, 0].set(Wb2[:, 0]).at[H // 2:, 1].set(Wsc2[:, 0])
    b2 = jnp.stack([bb2[0], bsc2[0]])
    nh = _node_heads_tc(z_t, W1, b1, W2, b2)
    binary_logit = nh[:, 0]
    score_frac = nh[:, 1]

    return (edge_logits, binary_logit, score_frac, z_t, z_s)


# speedup vs baseline: 1.0252x; 1.0252x over previous
import functools

import jax
import jax.numpy as jnp
from jax import lax
from jax.experimental import pallas as pl
from jax.experimental.pallas import tpu as pltpu
from jax.experimental.pallas import tpu_sc as plsc

H = 256
IN_DIM = 768
N_NODE = 10000
E_EDGE = 160000
EL_EDGE = 160000

NCORE = 2
NSUB = 16
NWORK = NCORE * NSUB
OWN = 320
TAB_ROWS = 328
TRASH = 320
RCH = 48
CAP = (E_EDGE // NWORK + 16 + RCH - 1) // RCH + 1
EPW = E_EDGE // NWORK
NGRP = EPW // 16
SLOT_WORDS = NWORK * NWORK * CAP * RCH
CADDR_MAX = NWORK * CAP
GAT_CHUNK = 40
GAT_PER_W = EL_EDGE // NWORK
GAT_ITERS = GAT_PER_W // GAT_CHUNK

_SC_MESH = dict(core_axis_name="c", subcore_axis_name="s")


def _iota16():
    return lax.broadcasted_iota(jnp.int32, (16,), 0)


def _gelu(x):
    return 0.5 * x * (1.0 + lax.erf(x * (2.0 ** -0.5)))


def _route_sc(src, dst):
    @functools.partial(
        pl.kernel,
        out_type=(jax.ShapeDtypeStruct((SLOT_WORDS,), jnp.int32),
                  jax.ShapeDtypeStruct((NWORK * NWORK,), jnp.int32),
                  jax.ShapeDtypeStruct((NWORK * N_NODE,), jnp.float32)),
        mesh=plsc.VectorSubcoreMesh(**_SC_MESH),
        scratch_types=[
            pltpu.VMEM((EPW + 16,), jnp.int32),
            pltpu.VMEM((EPW + 16,), jnp.int32),
            pltpu.VMEM((NWORK * RCH,), jnp.int32),
            pltpu.VMEM((NWORK + 16,), jnp.int32),
            pltpu.VMEM((NWORK + 16,), jnp.int32),
            pltpu.VMEM((16,), jnp.int32),
            pltpu.VMEM((N_NODE + 32,), jnp.float32),
        ],
    )
    def k(src_h, dst_h, slots_h, ncnt_h, cnt_h, sbuf, dbuf, ckbuf, pend, nch,
          stage, cbuf):
        c = lax.axis_index("c")
        w = lax.axis_index("s")
        wid = w * NCORE + c
        base_e = wid * EPW
        pltpu.sync_copy(src_h.at[pl.ds(base_e, EPW)], sbuf.at[pl.ds(0, EPW)])
        pltpu.sync_copy(dst_h.at[pl.ds(base_e, EPW)], dbuf.at[pl.ds(0, EPW)])
        iota = _iota16()
        zero = jnp.zeros((16,), jnp.int32)
        fz = jnp.zeros((16,), jnp.float32)
        for gg in range(3):
            pend[pl.ds(gg * 16, 16)] = zero
            nch[pl.ds(gg * 16, 16)] = zero

        def zc(r, carry):
            cbuf[pl.ds(r * 16, 16)] = fz
            return carry

        lax.fori_loop(0, (N_NODE + 32) // 16, zc, 0, unroll=False)

        def do_group(dstv, srcv, valid):
            ov = (dstv * 6554) >> 21
            locv = dstv - ov * OWN
            ov = jnp.where(valid, ov, 0)
            locv = jnp.where(valid, locv, TRASH)
            srcv = jnp.where(valid, srcv, 0)
            packv = srcv * 512 + locv
            dcnt = jnp.where(valid, dstv, N_NODE + 8)
            for l in range(16):
                o = ov[l]
                pk = packv[l]
                dd = dcnt[l]
                cv = cbuf[pl.ds(dd, 16)]
                cbuf[pl.ds(dd, 16)] = jnp.where(iota == 0, cv + 1.0, cv)
                pv = pend[pl.ds(o, 16)]
                p = pv[0]
                cbase = o * RCH + (p & ~jnp.int32(15))
                lane = p & 15
                v = ckbuf[pl.ds(cbase, 16)]
                ckbuf[pl.ds(cbase, 16)] = jnp.where(iota == lane, pk, v)
                p1 = p + 1

                @pl.when(p1 == RCH)
                def _flush():
                    nv = nch[pl.ds(o, 16)]
                    nc = nv[0]
                    off = ((wid * NWORK + o) * CAP + nc) * RCH
                    pltpu.sync_copy(ckbuf.at[pl.ds(o * RCH, RCH)],
                                    slots_h.at[pl.ds(off, RCH)])
                    nch[pl.ds(o, 16)] = jnp.where(iota == 0, nc + 1, nv)

                pend[pl.ds(o, 16)] = jnp.where(
                    iota == 0, jnp.where(p1 == RCH, 0, p1), pv)

        def step(g, carry):
            dstv = dbuf[pl.ds(g * 16, 16)]
            srcv = sbuf[pl.ds(g * 16, 16)]
            do_group(dstv, srcv, iota >= 0)
            return carry

        lax.fori_loop(0, NGRP, step, 0, unroll=False)
        dstv = dbuf[pl.ds(NGRP * 16, 16)]
        srcv = sbuf[pl.ds(NGRP * 16, 16)]
        do_group(dstv, srcv, iota < (EPW - NGRP * 16))

        for o in range(NWORK):
            pv = pend[pl.ds(o, 16)]
            p = pv[0]
            for gg in range(RCH // 16):
                gl = iota + gg * 16
                v = ckbuf[pl.ds(o * RCH + gg * 16, 16)]
                ckbuf[pl.ds(o * RCH + gg * 16, 16)] = jnp.where(
                    gl >= p, jnp.int32(TRASH), v)

            @pl.when(p > 0)
            def _flush():
                nv = nch[pl.ds(o, 16)]
                nc = nv[0]
                off = ((wid * NWORK + o) * CAP + nc) * RCH
                pltpu.sync_copy(ckbuf.at[pl.ds(o * RCH, RCH)],
                                slots_h.at[pl.ds(off, RCH)])
                nch[pl.ds(o, 16)] = jnp.where(iota == 0, nc + 1, nv)

        for gg in range(2):
            acc = jnp.zeros((16,), jnp.int32)
            for l in range(16):
                nv = nch[pl.ds(gg * 16 + l, 16)]
                acc = jnp.where(iota == l, nv[0], acc)
            stage[pl.ds(0, 16)] = acc
            pltpu.sync_copy(stage, ncnt_h.at[pl.ds(wid * NWORK + gg * 16, 16)])

        pltpu.sync_copy(cbuf.at[pl.ds(0, N_NODE)],
                        cnt_h.at[pl.ds(wid * N_NODE, N_NODE)])

    return k(src, dst)


def _segsum_sc(table, slots, ncnt):
    @functools.partial(
        pl.kernel,
        out_type=jax.ShapeDtypeStruct((N_NODE, H), jnp.float32),
        mesh=plsc.VectorSubcoreMesh(**_SC_MESH),
        scratch_types=[
            pltpu.VMEM((TAB_ROWS, H), jnp.float32),
            pltpu.VMEM((RCH, H), jnp.float32),
            pltpu.VMEM((RCH, H), jnp.float32),
            pltpu.VMEM((RCH + 16,), jnp.int32),
            pltpu.VMEM((RCH + 16,), jnp.int32),
            pltpu.VMEM((RCH,), jnp.int32),
            pltpu.VMEM((RCH,), jnp.int32),
            pltpu.VMEM((CADDR_MAX + 16,), jnp.int32),
            pltpu.VMEM((NWORK * NWORK + 16,), jnp.int32),
            pltpu.SemaphoreType.DMA,
            pltpu.SemaphoreType.DMA,
            pltpu.SemaphoreType.DMA,
            pltpu.SemaphoreType.DMA,
        ],
    )
    def k(table_h, slots_h, ncnt_h, agg_h,
          tab, rows_x, rows_y, pb_x, pb_y, sx, sy, caddr, ncb,
          lsem_x, lsem_y, gsem_x, gsem_y):
        c = lax.axis_index("c")
        w = lax.axis_index("s")
        o = w * NCORE + c
        iota = _iota16()
        fz = jnp.zeros((16,), jnp.float32)

        def zrow(r, carry):
            for jj in range(H // 16):
                tab[r, pl.ds(jj * 16, 16)] = fz
            return carry

        lax.fori_loop(0, TAB_ROWS, zrow, 0, unroll=False)

        pltpu.sync_copy(ncnt_h, ncb.at[pl.ds(0, NWORK * NWORK)])

        def build_w(wsc, t0):
            nv = ncb[pl.ds(wsc * NWORK + o, 16)]
            n_w = nv[0]

            def app(cc, t):
                addr = ((wsc * NWORK + o) * CAP + cc) * RCH
                b = t & ~jnp.int32(15)
                v = caddr[pl.ds(b, 16)]
                caddr[pl.ds(b, 16)] = jnp.where(iota == (t & 15), addr, v)
                return t + 1

            return lax.fori_loop(0, n_w, app, t0, unroll=False)

        t_total = jnp.int32(0)
        for wsc in range(NWORK):
            t_total = build_w(wsc, t_total)

        def accum(rows, pbuf):
            def arow(r, carry):
                pv = pbuf[pl.ds(r, 16)]
                loc = pv[0] & 511
                for jj in range(H // 16):
                    plsc.addupdate(tab.at[loc, pl.ds(jj * 16, 16)],
                                   rows[r, pl.ds(jj * 16, 16)])
                return carry

            lax.fori_loop(0, RCH, arow, 0, unroll=False)

        def unpack(pbuf, sref):
            for gg in range(RCH // 16):
                pv = pbuf[pl.ds(gg * 16, 16)]
                sref[pl.ds(gg * 16, 16)] = lax.shift_right_logical(pv, 9)

        def chunk_addr(j):
            av = caddr[pl.ds(j, 16)]
            return pl.multiple_of(av[0], RCH)

        def stage(j, pbufA, sA, rowsA, lsemA, gsemA, pbufB, sB, rowsB, lsemB, gsemB):
            pltpu.make_async_copy(slots_h.at[pl.ds(0, RCH)],
                                  pbufA.at[pl.ds(0, RCH)], lsemA).wait()
            unpack(pbufA, sA)
            pltpu.async_copy(table_h.at[sA], rowsA, gsemA)

            @pl.when(j > 0)
            def _acc_prev():
                pltpu.make_async_copy(table_h.at[sB], rowsB, gsemB).wait()
                accum(rowsB, pbufB)

            @pl.when(j + 1 < t_total)
            def _next_load():
                pltpu.async_copy(slots_h.at[pl.ds(chunk_addr(j + 1), RCH)],
                                 pbufB.at[pl.ds(0, RCH)], lsemB)

        @pl.when(t_total > 0)
        def _prologue():
            pltpu.async_copy(slots_h.at[pl.ds(chunk_addr(jnp.int32(0)), RCH)],
                             pb_x.at[pl.ds(0, RCH)], lsem_x)

        def pipe(j, carry):
            @pl.when((j & 1) == 0)
            def _x():
                stage(j, pb_x, sx, rows_x, lsem_x, gsem_x,
                      pb_y, sy, rows_y, lsem_y, gsem_y)

            @pl.when((j & 1) == 1)
            def _y():
                stage(j, pb_y, sy, rows_y, lsem_y, gsem_y,
                      pb_x, sx, rows_x, lsem_x, gsem_x)

            return carry

        lax.fori_loop(0, t_total, pipe, 0, unroll=False)

        @pl.when((t_total > 0) & ((t_total & 1) == 1))
        def _epi_x():
            pltpu.make_async_copy(table_h.at[sx], rows_x, gsem_x).wait()
            accum(rows_x, pb_x)

        @pl.when((t_total > 0) & ((t_total & 1) == 0))
        def _epi_y():
            pltpu.make_async_copy(table_h.at[sy], rows_y, gsem_y).wait()
            accum(rows_y, pb_y)

        @pl.when(o < NWORK - 1)
        def _wb():
            pltpu.sync_copy(tab.at[pl.ds(0, OWN)], agg_h.at[pl.ds(o * OWN, OWN)])

        @pl.when(o == NWORK - 1)
        def _wb_last():
            nlast = N_NODE - (NWORK - 1) * OWN
            pltpu.sync_copy(tab.at[pl.ds(0, nlast)],
                            agg_h.at[pl.ds((NWORK - 1) * OWN, nlast)])

    return k(table, slots, ncnt)


def _gather2_sc(t0, t1, i0, i1):
    @functools.partial(
        pl.kernel,
        out_type=(jax.ShapeDtypeStruct((EL_EDGE, H), jnp.float32),
                  jax.ShapeDtypeStruct((EL_EDGE, H), jnp.float32)),
        mesh=plsc.VectorSubcoreMesh(**_SC_MESH),
        scratch_types=[
            pltpu.VMEM((GAT_CHUNK,), jnp.int32),
            pltpu.VMEM((GAT_CHUNK,), jnp.int32),
            pltpu.VMEM((GAT_CHUNK, H), jnp.float32),
            pltpu.VMEM((GAT_CHUNK, H), jnp.float32),
            pltpu.SemaphoreType.DMA,
            pltpu.SemaphoreType.DMA,
        ],
    )
    def k(t0_h, t1_h, i0_h, i1_h, g0_h, g1_h, i0_v, i1_v, r0_v, r1_v, s0, s1):
        c = lax.axis_index("c")
        w = lax.axis_index("s")
        wid = w * NCORE + c
        base0 = wid * GAT_PER_W

        def step(i, carry):
            base = base0 + i * GAT_CHUNK
            pltpu.sync_copy(i0_h.at[pl.ds(base, GAT_CHUNK)], i0_v)
            pltpu.sync_copy(i1_h.at[pl.ds(base, GAT_CHUNK)], i1_v)
            cp0 = pltpu.async_copy(t0_h.at[i0_v], r0_v, s0)
            cp1 = pltpu.async_copy(t1_h.at[i1_v], r1_v, s1)
            cp0.wait()
            cp1.wait()
            pltpu.sync_copy(r0_v, g0_h.at[pl.ds(base, GAT_CHUNK)])
            pltpu.sync_copy(r1_v, g1_h.at[pl.ds(base, GAT_CHUNK)])
            return carry

        lax.fori_loop(0, GAT_ITERS, step, 0)

    return k(t0, t1, i0, i1)


def _proj_tc(x, W, b, bm=1000):
    M, K = x.shape
    N = W.shape[1]

    def body(x_ref, w_ref, b_ref, o_ref):
        acc = jnp.dot(x_ref[...], w_ref[...], preferred_element_type=jnp.float32)
        o_ref[...] = _gelu(acc + b_ref[...])

    return pl.pallas_call(
        body,
        grid=(M // bm,),
        in_specs=[pl.BlockSpec((bm, K), lambda i: (i, 0)),
                  pl.BlockSpec((K, N), lambda i: (0, 0)),
                  pl.BlockSpec((1, N), lambda i: (0, 0))],
        out_specs=pl.BlockSpec((bm, N), lambda i: (i, 0)),
        out_shape=jax.ShapeDtypeStruct((M, N), jnp.float32),
    )(x, W, b.reshape(1, N))


def _combine_tc(agg, cnt_part, z_dst, Wl, Wr, b, bm=1000):
    M = agg.shape[0]

    def body(a_ref, c_ref, z_ref, wl_ref, wr_ref, b_ref, o_ref):
        cntc = jnp.sum(c_ref[...], axis=1)[:, None]
        mean = a_ref[...] / jnp.maximum(cntc, 1.0)
        acc = jnp.dot(mean, wl_ref[...], preferred_element_type=jnp.float32)
        acc += jnp.dot(z_ref[...], wr_ref[...], preferred_element_type=jnp.float32)
        o_ref[...] = _gelu(acc + b_ref[...])

    return pl.pallas_call(
        body,
        grid=(M // bm,),
        in_specs=[pl.BlockSpec((bm, H), lambda i: (i, 0)),
                  pl.BlockSpec((bm, NWORK), lambda i: (i, 0)),
                  pl.BlockSpec((bm, H), lambda i: (i, 0)),
                  pl.BlockSpec((H, H), lambda i: (0, 0)),
                  pl.BlockSpec((H, H), lambda i: (0, 0)),
                  pl.BlockSpec((1, H), lambda i: (0, 0))],
        out_specs=pl.BlockSpec((bm, H), lambda i: (i, 0)),
        out_shape=jax.ShapeDtypeStruct((M, H), jnp.float32),
    )(agg, cnt_part, z_dst, Wl, Wr, b.reshape(1, H))


def _edge_head_tc(zt_g, zs_g, Wa, Wb, Wc, b1, W2, b2, bm=1000):
    M = zt_g.shape[0]
    NO = W2.shape[1]

    def body(zt_ref, zs_ref, wa_ref, wb_ref, wc_ref, b1_ref, w2_ref, b2_ref, o_ref):
        zt = zt_ref[...]
        zs = zs_ref[...]
        d = jnp.abs(zt - zs)
        acc = jnp.dot(zt, wa_ref[...], preferred_element_type=jnp.float32)
        acc += jnp.dot(zs, wb_ref[...], preferred_element_type=jnp.float32)
        acc += jnp.dot(d, wc_ref[...], preferred_element_type=jnp.float32)
        g = _gelu(acc + b1_ref[...])
        o_ref[...] = jnp.dot(g, w2_ref[...], preferred_element_type=jnp.float32) + b2_ref[...]

    return pl.pallas_call(
        body,
        grid=(M // bm,),
        in_specs=[pl.BlockSpec((bm, H), lambda i: (i, 0)),
                  pl.BlockSpec((bm, H), lambda i: (i, 0)),
                  pl.BlockSpec((H, H), lambda i: (0, 0)),
                  pl.BlockSpec((H, H), lambda i: (0, 0)),
                  pl.BlockSpec((H, H), lambda i: (0, 0)),
                  pl.BlockSpec((1, H), lambda i: (0, 0)),
                  pl.BlockSpec((H, NO), lambda i: (0, 0)),
                  pl.BlockSpec((1, NO), lambda i: (0, 0))],
        out_specs=pl.BlockSpec((bm, NO), lambda i: (i, 0)),
        out_shape=jax.ShapeDtypeStruct((M, NO), jnp.float32),
    )(zt_g, zs_g, Wa, Wb, Wc, b1.reshape(1, H), W2, b2.reshape(1, NO))


def _node_heads_tc(z, W1, b1, W2, b2, bm=1000):
    M = z.shape[0]

    def body(z_ref, w1_ref, b1_ref, w2_ref, b2_ref, o_ref):
        h = _gelu(jnp.dot(z_ref[...], w1_ref[...], preferred_element_type=jnp.float32)
                  + b1_ref[...])
        o = jnp.dot(h, w2_ref[...], preferred_element_type=jnp.float32) + b2_ref[...]
        col = lax.broadcasted_iota(jnp.int32, o.shape, 1)
        o_ref[...] = jnp.where(col == 1, jax.nn.sigmoid(o), o)

    return pl.pallas_call(
        body,
        grid=(M // bm,),
        in_specs=[pl.BlockSpec((bm, H), lambda i: (i, 0)),
                  pl.BlockSpec((H, H), lambda i: (0, 0)),
                  pl.BlockSpec((1, H), lambda i: (0, 0)),
                  pl.BlockSpec((H, 2), lambda i: (0, 0)),
                  pl.BlockSpec((1, 2), lambda i: (0, 0))],
        out_specs=pl.BlockSpec((bm, 2), lambda i: (i, 0)),
        out_shape=jax.ShapeDtypeStruct((M, 2), jnp.float32),
    )(z, W1, b1.reshape(1, H), W2, b2.reshape(1, 2))


def kernel(x_transcript, x_symptom, edge_index_ts, edge_index_st, edge_label_index,
           Wt_proj, bt_proj, Ws_proj, bs_proj,
           Wl0_ts, bl0_ts, Wr0_ts, Wl0_st, bl0_st, Wr0_st,
           Wl1_ts, bl1_ts, Wr1_ts, Wl1_st, bl1_st, Wr1_st,
           We1, be1, We2, be2, Wb1, bb1, Wb2, bb2, Wsc1, bsc1, Wsc2, bsc2):
    ei_ts = edge_index_ts.astype(jnp.int32)
    ei_st = edge_index_st.astype(jnp.int32)
    eli = edge_label_index.astype(jnp.int32)

    slots_ts, ncnt_ts, cntp_ts = _route_sc(ei_ts[0], ei_ts[1])
    slots_st, ncnt_st, cntp_st = _route_sc(ei_st[0], ei_st[1])
    cntp_ts = cntp_ts.reshape(NWORK, N_NODE).T
    cntp_st = cntp_st.reshape(NWORK, N_NODE).T

    z_t = _proj_tc(x_transcript, Wt_proj, bt_proj)
    z_s = _proj_tc(x_symptom, Ws_proj, bs_proj)

    layers = ((Wl0_ts, bl0_ts, Wr0_ts, Wl0_st, bl0_st, Wr0_st),
              (Wl1_ts, bl1_ts, Wr1_ts, Wl1_st, bl1_st, Wr1_st))
    for (Wl_ts, bl_ts, Wr_ts, Wl_st, bl_st, Wr_st) in layers:
        agg_s = _segsum_sc(z_t, slots_ts, ncnt_ts)
        agg_t = _segsum_sc(z_s, slots_st, ncnt_st)
        new_s = _combine_tc(agg_s, cntp_ts, z_s, Wl_ts, Wr_ts, bl_ts)
        new_t = _combine_tc(agg_t, cntp_st, z_t, Wl_st, Wr_st, bl_st)
        z_t, z_s = new_t, new_s

    zt_g, zs_g = _gather2_sc(z_t, z_s, eli[0], eli[1])

    Wa, Wb, Wc = We1[:H], We1[H:2 * H], We1[2 * H:]
    edge_logits = _edge_head_tc(zt_g, zs_g, Wa, Wb, Wc, be1, We2, be2)

    W1 = jnp.concatenate([Wb1, Wsc1], axis=1)
    b1 = jnp.concatenate([bb1, bsc1])
    W2 = jnp.zeros((H, 2), jnp.float32)
    W2 = W2.at[:H // 2, 0].set(Wb2[:, 0]).at[H // 2:, 1].set(Wsc2[:, 0])
    b2 = jnp.stack([bb2[0], bsc2[0]])
    nh = _node_heads_tc(z_t, W1, b1, W2, b2)
    binary_logit = nh[:, 0]
    score_frac = nh[:, 1]

    return (edge_logits, binary_logit, score_frac, z_t, z_s)


# speedup vs baseline: 1.0269x; 1.0016x over previous
import functools

import jax
import jax.numpy as jnp
from jax import lax
from jax.experimental import pallas as pl
from jax.experimental.pallas import tpu as pltpu
from jax.experimental.pallas import tpu_sc as plsc

H = 256
IN_DIM = 768
N_NODE = 10000
E_EDGE = 160000
EL_EDGE = 160000

NCORE = 2
NSUB = 16
NWORK = NCORE * NSUB
OWN = 320
TAB_ROWS = 328
TRASH = 320
RCH = 48
CAP = (E_EDGE // NWORK + 16 + RCH - 1) // RCH + 1
EPW = E_EDGE // NWORK
NGRP = EPW // 16
SLOT_WORDS = NWORK * NWORK * CAP * RCH
CADDR_MAX = NWORK * CAP
GAT_CHUNK = 40
GAT_PER_W = EL_EDGE // NWORK
GAT_ITERS = GAT_PER_W // GAT_CHUNK

_SC_MESH = dict(core_axis_name="c", subcore_axis_name="s")


def _iota16():
    return lax.broadcasted_iota(jnp.int32, (16,), 0)


def _gelu(x):
    return 0.5 * x * (1.0 + lax.erf(x * (2.0 ** -0.5)))


def _route_sc(src, dst):
    @functools.partial(
        pl.kernel,
        out_type=(jax.ShapeDtypeStruct((SLOT_WORDS,), jnp.int32),
                  jax.ShapeDtypeStruct((NWORK * NWORK,), jnp.int32),
                  jax.ShapeDtypeStruct((NWORK * N_NODE,), jnp.float32)),
        mesh=plsc.VectorSubcoreMesh(**_SC_MESH),
        scratch_types=[
            pltpu.VMEM((EPW + 16,), jnp.int32),
            pltpu.VMEM((EPW + 16,), jnp.int32),
            pltpu.VMEM((NWORK * RCH,), jnp.int32),
            pltpu.VMEM((NWORK + 16,), jnp.int32),
            pltpu.VMEM((NWORK + 16,), jnp.int32),
            pltpu.VMEM((16,), jnp.int32),
            pltpu.VMEM((N_NODE + 32,), jnp.float32),
        ],
    )
    def k(src_h, dst_h, slots_h, ncnt_h, cnt_h, sbuf, dbuf, ckbuf, pend, nch,
          stage, cbuf):
        c = lax.axis_index("c")
        w = lax.axis_index("s")
        wid = w * NCORE + c
        base_e = wid * EPW
        pltpu.sync_copy(src_h.at[pl.ds(base_e, EPW)], sbuf.at[pl.ds(0, EPW)])
        pltpu.sync_copy(dst_h.at[pl.ds(base_e, EPW)], dbuf.at[pl.ds(0, EPW)])
        iota = _iota16()
        zero = jnp.zeros((16,), jnp.int32)
        fz = jnp.zeros((16,), jnp.float32)
        for gg in range(3):
            pend[pl.ds(gg * 16, 16)] = zero
            nch[pl.ds(gg * 16, 16)] = zero

        def zc(r, carry):
            cbuf[pl.ds(r * 16, 16)] = fz
            return carry

        lax.fori_loop(0, (N_NODE + 32) // 16, zc, 0, unroll=False)

        def do_group(dstv, srcv, valid):
            ov = (dstv * 6554) >> 21
            locv = dstv - ov * OWN
            ov = jnp.where(valid, ov, 0)
            locv = jnp.where(valid, locv, TRASH)
            srcv = jnp.where(valid, srcv, 0)
            packv = srcv * 512 + locv
            dcnt = jnp.where(valid, dstv, N_NODE + 8)
            for l in range(16):
                o = ov[l]
                pk = packv[l]
                dd = dcnt[l]
                cv = cbuf[pl.ds(dd, 16)]
                cbuf[pl.ds(dd, 16)] = jnp.where(iota == 0, cv + 1.0, cv)
                pv = pend[pl.ds(o, 16)]
                p = pv[0]
                cbase = o * RCH + (p & ~jnp.int32(15))
                lane = p & 15
                v = ckbuf[pl.ds(cbase, 16)]
                ckbuf[pl.ds(cbase, 16)] = jnp.where(iota == lane, pk, v)
                p1 = p + 1

                @pl.when(p1 == RCH)
                def _flush():
                    nv = nch[pl.ds(o, 16)]
                    nc = nv[0]
                    off = ((wid * NWORK + o) * CAP + nc) * RCH
                    pltpu.sync_copy(ckbuf.at[pl.ds(o * RCH, RCH)],
                                    slots_h.at[pl.ds(off, RCH)])
                    nch[pl.ds(o, 16)] = jnp.where(iota == 0, nc + 1, nv)

                pend[pl.ds(o, 16)] = jnp.where(
                    iota == 0, jnp.where(p1 == RCH, 0, p1), pv)

        def step(g, carry):
            dstv = dbuf[pl.ds(g * 16, 16)]
            srcv = sbuf[pl.ds(g * 16, 16)]
            do_group(dstv, srcv, iota >= 0)
            return carry

        lax.fori_loop(0, NGRP, step, 0, unroll=False)
        dstv = dbuf[pl.ds(NGRP * 16, 16)]
        srcv = sbuf[pl.ds(NGRP * 16, 16)]
        do_group(dstv, srcv, iota < (EPW - NGRP * 16))

        for o in range(NWORK):
            pv = pend[pl.ds(o, 16)]
            p = pv[0]
            for gg in range(RCH // 16):
                gl = iota + gg * 16
                v = ckbuf[pl.ds(o * RCH + gg * 16, 16)]
                ckbuf[pl.ds(o * RCH + gg * 16, 16)] = jnp.where(
                    gl >= p, jnp.int32(TRASH), v)

            @pl.when(p > 0)
            def _flush():
                nv = nch[pl.ds(o, 16)]
                nc = nv[0]
                off = ((wid * NWORK + o) * CAP + nc) * RCH
                pltpu.sync_copy(ckbuf.at[pl.ds(o * RCH, RCH)],
                                slots_h.at[pl.ds(off, RCH)])
                nch[pl.ds(o, 16)] = jnp.where(iota == 0, nc + 1, nv)

        for gg in range(2):
            acc = jnp.zeros((16,), jnp.int32)
            for l in range(16):
                nv = nch[pl.ds(gg * 16 + l, 16)]
                acc = jnp.where(iota == l, nv[0], acc)
            stage[pl.ds(0, 16)] = acc
            pltpu.sync_copy(stage, ncnt_h.at[pl.ds(wid * NWORK + gg * 16, 16)])

        pltpu.sync_copy(cbuf.at[pl.ds(0, N_NODE)],
                        cnt_h.at[pl.ds(wid * N_NODE, N_NODE)])

    return k(src, dst)


def _segsum_sc(table, slots, ncnt):
    @functools.partial(
        pl.kernel,
        out_type=jax.ShapeDtypeStruct((N_NODE, H), jnp.float32),
        mesh=plsc.VectorSubcoreMesh(**_SC_MESH),
        scratch_types=[
            pltpu.VMEM((TAB_ROWS, H), jnp.float32),
            pltpu.VMEM((RCH, H), jnp.float32),
            pltpu.VMEM((RCH, H), jnp.float32),
            pltpu.VMEM((RCH + 16,), jnp.int32),
            pltpu.VMEM((RCH + 16,), jnp.int32),
            pltpu.VMEM((RCH,), jnp.int32),
            pltpu.VMEM((RCH,), jnp.int32),
            pltpu.VMEM((CADDR_MAX + 16,), jnp.int32),
            pltpu.VMEM((NWORK * NWORK + 16,), jnp.int32),
            pltpu.SemaphoreType.DMA,
            pltpu.SemaphoreType.DMA,
            pltpu.SemaphoreType.DMA,
            pltpu.SemaphoreType.DMA,
        ],
    )
    def k(table_h, slots_h, ncnt_h, agg_h,
          tab, rows_x, rows_y, pb_x, pb_y, sx, sy, caddr, ncb,
          lsem_x, lsem_y, gsem_x, gsem_y):
        c = lax.axis_index("c")
        w = lax.axis_index("s")
        o = w * NCORE + c
        iota = _iota16()
        fz = jnp.zeros((16,), jnp.float32)

        def zrow(r, carry):
            for jj in range(H // 16):
                tab[r, pl.ds(jj * 16, 16)] = fz
            return carry

        lax.fori_loop(0, TAB_ROWS, zrow, 0, unroll=False)

        pltpu.sync_copy(ncnt_h, ncb.at[pl.ds(0, NWORK * NWORK)])

        def build_w(wsc, t0):
            nv = ncb[pl.ds(wsc * NWORK + o, 16)]
            n_w = nv[0]

            def app(cc, t):
                addr = ((wsc * NWORK + o) * CAP + cc) * RCH
                b = t & ~jnp.int32(15)
                v = caddr[pl.ds(b, 16)]
                caddr[pl.ds(b, 16)] = jnp.where(iota == (t & 15), addr, v)
                return t + 1

            return lax.fori_loop(0, n_w, app, t0, unroll=False)

        t_total = jnp.int32(0)
        for wsc in range(NWORK):
            t_total = build_w(wsc, t_total)

        def accum(rows, pbuf):
            def arow(r, carry):
                pv = pbuf[pl.ds(r, 16)]
                loc = pv[0] & 511
                for jj in range(H // 16):
                    plsc.addupdate(tab.at[loc, pl.ds(jj * 16, 16)],
                                   rows[r, pl.ds(jj * 16, 16)])
                return carry

            lax.fori_loop(0, RCH, arow, 0, unroll=4)

        def unpack(pbuf, sref):
            for gg in range(RCH // 16):
                pv = pbuf[pl.ds(gg * 16, 16)]
                sref[pl.ds(gg * 16, 16)] = lax.shift_right_logical(pv, 9)

        def chunk_addr(j):
            av = caddr[pl.ds(j, 16)]
            return pl.multiple_of(av[0], RCH)

        def stage(j, pbufA, sA, rowsA, lsemA, gsemA, pbufB, sB, rowsB, lsemB, gsemB):
            pltpu.make_async_copy(slots_h.at[pl.ds(0, RCH)],
                                  pbufA.at[pl.ds(0, RCH)], lsemA).wait()
            unpack(pbufA, sA)
            pltpu.async_copy(table_h.at[sA], rowsA, gsemA)

            @pl.when(j > 0)
            def _acc_prev():
                pltpu.make_async_copy(table_h.at[sB], rowsB, gsemB).wait()
                accum(rowsB, pbufB)

            @pl.when(j + 1 < t_total)
            def _next_load():
                pltpu.async_copy(slots_h.at[pl.ds(chunk_addr(j + 1), RCH)],
                                 pbufB.at[pl.ds(0, RCH)], lsemB)

        @pl.when(t_total > 0)
        def _prologue():
            pltpu.async_copy(slots_h.at[pl.ds(chunk_addr(jnp.int32(0)), RCH)],
                             pb_x.at[pl.ds(0, RCH)], lsem_x)

        def pipe(j, carry):
            @pl.when((j & 1) == 0)
            def _x():
                stage(j, pb_x, sx, rows_x, lsem_x, gsem_x,
                      pb_y, sy, rows_y, lsem_y, gsem_y)

            @pl.when((j & 1) == 1)
            def _y():
                stage(j, pb_y, sy, rows_y, lsem_y, gsem_y,
                      pb_x, sx, rows_x, lsem_x, gsem_x)

            return carry

        lax.fori_loop(0, t_total, pipe, 0, unroll=False)

        @pl.when((t_total > 0) & ((t_total & 1) == 1))
        def _epi_x():
            pltpu.make_async_copy(table_h.at[sx], rows_x, gsem_x).wait()
            accum(rows_x, pb_x)

        @pl.when((t_total > 0) & ((t_total & 1) == 0))
        def _epi_y():
            pltpu.make_async_copy(table_h.at[sy], rows_y, gsem_y).wait()
            accum(rows_y, pb_y)

        @pl.when(o < NWORK - 1)
        def _wb():
            pltpu.sync_copy(tab.at[pl.ds(0, OWN)], agg_h.at[pl.ds(o * OWN, OWN)])

        @pl.when(o == NWORK - 1)
        def _wb_last():
            nlast = N_NODE - (NWORK - 1) * OWN
            pltpu.sync_copy(tab.at[pl.ds(0, nlast)],
                            agg_h.at[pl.ds((NWORK - 1) * OWN, nlast)])

    return k(table, slots, ncnt)


def _gather2_sc(t0, t1, i0, i1):
    @functools.partial(
        pl.kernel,
        out_type=(jax.ShapeDtypeStruct((EL_EDGE, H), jnp.float32),
                  jax.ShapeDtypeStruct((EL_EDGE, H), jnp.float32)),
        mesh=plsc.VectorSubcoreMesh(**_SC_MESH),
        scratch_types=[
            pltpu.VMEM((GAT_CHUNK,), jnp.int32),
            pltpu.VMEM((GAT_CHUNK,), jnp.int32),
            pltpu.VMEM((GAT_CHUNK, H), jnp.float32),
            pltpu.VMEM((GAT_CHUNK, H), jnp.float32),
            pltpu.SemaphoreType.DMA,
            pltpu.SemaphoreType.DMA,
        ],
    )
    def k(t0_h, t1_h, i0_h, i1_h, g0_h, g1_h, i0_v, i1_v, r0_v, r1_v, s0, s1):
        c = lax.axis_index("c")
        w = lax.axis_index("s")
        wid = w * NCORE + c
        base0 = wid * GAT_PER_W

        def step(i, carry):
            base = base0 + i * GAT_CHUNK
            pltpu.sync_copy(i0_h.at[pl.ds(base, GAT_CHUNK)], i0_v)
            pltpu.sync_copy(i1_h.at[pl.ds(base, GAT_CHUNK)], i1_v)
            cp0 = pltpu.async_copy(t0_h.at[i0_v], r0_v, s0)
            cp1 = pltpu.async_copy(t1_h.at[i1_v], r1_v, s1)
            cp0.wait()
            cp1.wait()
            pltpu.sync_copy(r0_v, g0_h.at[pl.ds(base, GAT_CHUNK)])
            pltpu.sync_copy(r1_v, g1_h.at[pl.ds(base, GAT_CHUNK)])
            return carry

        lax.fori_loop(0, GAT_ITERS, step, 0)

    return k(t0, t1, i0, i1)


def _proj_tc(x, W, b, bm=1000):
    M, K = x.shape
    N = W.shape[1]

    def body(x_ref, w_ref, b_ref, o_ref):
        acc = jnp.dot(x_ref[...], w_ref[...], preferred_element_type=jnp.float32)
        o_ref[...] = _gelu(acc + b_ref[...])

    return pl.pallas_call(
        body,
        grid=(M // bm,),
        in_specs=[pl.BlockSpec((bm, K), lambda i: (i, 0)),
                  pl.BlockSpec((K, N), lambda i: (0, 0)),
                  pl.BlockSpec((1, N), lambda i: (0, 0))],
        out_specs=pl.BlockSpec((bm, N), lambda i: (i, 0)),
        out_shape=jax.ShapeDtypeStruct((M, N), jnp.float32),
    )(x, W, b.reshape(1, N))


def _combine_tc(agg, cnt_part, z_dst, Wl, Wr, b, bm=1000):
    M = agg.shape[0]

    def body(a_ref, c_ref, z_ref, wl_ref, wr_ref, b_ref, o_ref):
        cntc = jnp.sum(c_ref[...], axis=1)[:, None]
        mean = a_ref[...] / jnp.maximum(cntc, 1.0)
        acc = jnp.dot(mean, wl_ref[...], preferred_element_type=jnp.float32)
        acc += jnp.dot(z_ref[...], wr_ref[...], preferred_element_type=jnp.float32)
        o_ref[...] = _gelu(acc + b_ref[...])

    return pl.pallas_call(
        body,
        grid=(M // bm,),
        in_specs=[pl.BlockSpec((bm, H), lambda i: (i, 0)),
                  pl.BlockSpec((bm, NWORK), lambda i: (i, 0)),
                  pl.BlockSpec((bm, H), lambda i: (i, 0)),
                  pl.BlockSpec((H, H), lambda i: (0, 0)),
                  pl.BlockSpec((H, H), lambda i: (0, 0)),
                  pl.BlockSpec((1, H), lambda i: (0, 0))],
        out_specs=pl.BlockSpec((bm, H), lambda i: (i, 0)),
        out_shape=jax.ShapeDtypeStruct((M, H), jnp.float32),
    )(agg, cnt_part, z_dst, Wl, Wr, b.reshape(1, H))


def _edge_head_tc(zt_g, zs_g, Wa, Wb, Wc, b1, W2, b2, bm=1000):
    M = zt_g.shape[0]
    NO = W2.shape[1]

    def body(zt_ref, zs_ref, wa_ref, wb_ref, wc_ref, b1_ref, w2_ref, b2_ref, o_ref):
        zt = zt_ref[...]
        zs = zs_ref[...]
        d = jnp.abs(zt - zs)
        acc = jnp.dot(zt, wa_ref[...], preferred_element_type=jnp.float32)
        acc += jnp.dot(zs, wb_ref[...], preferred_element_type=jnp.float32)
        acc += jnp.dot(d, wc_ref[...], preferred_element_type=jnp.float32)
        g = _gelu(acc + b1_ref[...])
        o_ref[...] = jnp.dot(g, w2_ref[...], preferred_element_type=jnp.float32) + b2_ref[...]

    return pl.pallas_call(
        body,
        grid=(M // bm,),
        in_specs=[pl.BlockSpec((bm, H), lambda i: (i, 0)),
                  pl.BlockSpec((bm, H), lambda i: (i, 0)),
                  pl.BlockSpec((H, H), lambda i: (0, 0)),
                  pl.BlockSpec((H, H), lambda i: (0, 0)),
                  pl.BlockSpec((H, H), lambda i: (0, 0)),
                  pl.BlockSpec((1, H), lambda i: (0, 0)),
                  pl.BlockSpec((H, NO), lambda i: (0, 0)),
                  pl.BlockSpec((1, NO), lambda i: (0, 0))],
        out_specs=pl.BlockSpec((bm, NO), lambda i: (i, 0)),
        out_shape=jax.ShapeDtypeStruct((M, NO), jnp.float32),
    )(zt_g, zs_g, Wa, Wb, Wc, b1.reshape(1, H), W2, b2.reshape(1, NO))


def _node_heads_tc(z, W1, b1, W2, b2, bm=1000):
    M = z.shape[0]

    def body(z_ref, w1_ref, b1_ref, w2_ref, b2_ref, o_ref):
        h = _gelu(jnp.dot(z_ref[...], w1_ref[...], preferred_element_type=jnp.float32)
                  + b1_ref[...])
        o = jnp.dot(h, w2_ref[...], preferred_element_type=jnp.float32) + b2_ref[...]
        col = lax.broadcasted_iota(jnp.int32, o.shape, 1)
        o_ref[...] = jnp.where(col == 1, jax.nn.sigmoid(o), o)

    return pl.pallas_call(
        body,
        grid=(M // bm,),
        in_specs=[pl.BlockSpec((bm, H), lambda i: (i, 0)),
                  pl.BlockSpec((H, H), lambda i: (0, 0)),
                  pl.BlockSpec((1, H), lambda i: (0, 0)),
                  pl.BlockSpec((H, 2), lambda i: (0, 0)),
                  pl.BlockSpec((1, 2), lambda i: (0, 0))],
        out_specs=pl.BlockSpec((bm, 2), lambda i: (i, 0)),
        out_shape=jax.ShapeDtypeStruct((M, 2), jnp.float32),
    )(z, W1, b1.reshape(1, H), W2, b2.reshape(1, 2))


def kernel(x_transcript, x_symptom, edge_index_ts, edge_index_st, edge_label_index,
           Wt_proj, bt_proj, Ws_proj, bs_proj,
           Wl0_ts, bl0_ts, Wr0_ts, Wl0_st, bl0_st, Wr0_st,
           Wl1_ts, bl1_ts, Wr1_ts, Wl1_st, bl1_st, Wr1_st,
           We1, be1, We2, be2, Wb1, bb1, Wb2, bb2, Wsc1, bsc1, Wsc2, bsc2):
    ei_ts = edge_index_ts.astype(jnp.int32)
    ei_st = edge_index_st.astype(jnp.int32)
    eli = edge_label_index.astype(jnp.int32)

    slots_ts, ncnt_ts, cntp_ts = _route_sc(ei_ts[0], ei_ts[1])
    slots_st, ncnt_st, cntp_st = _route_sc(ei_st[0], ei_st[1])
    cntp_ts = cntp_ts.reshape(NWORK, N_NODE).T
    cntp_st = cntp_st.reshape(NWORK, N_NODE).T

    z_t = _proj_tc(x_transcript, Wt_proj, bt_proj)
    z_s = _proj_tc(x_symptom, Ws_proj, bs_proj)

    layers = ((Wl0_ts, bl0_ts, Wr0_ts, Wl0_st, bl0_st, Wr0_st),
              (Wl1_ts, bl1_ts, Wr1_ts, Wl1_st, bl1_st, Wr1_st))
    for (Wl_ts, bl_ts, Wr_ts, Wl_st, bl_st, Wr_st) in layers:
        agg_s = _segsum_sc(z_t, slots_ts, ncnt_ts)
        agg_t = _segsum_sc(z_s, slots_st, ncnt_st)
        new_s = _combine_tc(agg_s, cntp_ts, z_s, Wl_ts, Wr_ts, bl_ts)
        new_t = _combine_tc(agg_t, cntp_st, z_t, Wl_st, Wr_st, bl_st)
        z_t, z_s = new_t, new_s

    zt_g, zs_g = _gather2_sc(z_t, z_s, eli[0], eli[1])

    Wa, Wb, Wc = We1[:H], We1[H:2 * H], We1[2 * H:]
    edge_logits = _edge_head_tc(zt_g, zs_g, Wa, Wb, Wc, be1, We2, be2)

    W1 = jnp.concatenate([Wb1, Wsc1], axis=1)
    b1 = jnp.concatenate([bb1, bsc1])
    W2 = jnp.zeros((H, 2), jnp.float32)
    W2 = W2.at[:H // 2, 0].set(Wb2[:, 0]).at[H // 2:, 1].set(Wsc2[:, 0])
    b2 = jnp.stack([bb2[0], bsc2[0]])
    nh = _node_heads_tc(z_t, W1, b1, W2, b2)
    binary_logit = nh[:, 0]
    score_frac = nh[:, 1]

    return (edge_logits, binary_logit, score_frac, z_t, z_s)


# speedup vs baseline: 1.0377x; 1.0106x over previous
import functools

import jax
import jax.numpy as jnp
from jax import lax
from jax.experimental import pallas as pl
from jax.experimental.pallas import tpu as pltpu
from jax.experimental.pallas import tpu_sc as plsc

H = 256
IN_DIM = 768
N_NODE = 10000
E_EDGE = 160000
EL_EDGE = 160000

NCORE = 2
NSUB = 16
NWORK = NCORE * NSUB
OWN = 320
TAB_ROWS = 328
TRASH = 320
RCH = 48
CAP = (E_EDGE // NWORK + 16 + RCH - 1) // RCH + 1
EPW = E_EDGE // NWORK
NGRP = EPW // 16
SLOT_WORDS = NWORK * NWORK * CAP * RCH
CADDR_MAX = NWORK * CAP
GAT_CHUNK = 40
GAT_PER_W = EL_EDGE // NWORK
GAT_ITERS = GAT_PER_W // GAT_CHUNK

_SC_MESH = dict(core_axis_name="c", subcore_axis_name="s")


def _iota16():
    return lax.broadcasted_iota(jnp.int32, (16,), 0)


def _gelu(x):
    return 0.5 * x * (1.0 + lax.erf(x * (2.0 ** -0.5)))


def _route_sc(src, dst):
    @functools.partial(
        pl.kernel,
        out_type=(jax.ShapeDtypeStruct((SLOT_WORDS,), jnp.int32),
                  jax.ShapeDtypeStruct((NWORK * NWORK,), jnp.int32),
                  jax.ShapeDtypeStruct((NWORK * N_NODE,), jnp.float32)),
        mesh=plsc.VectorSubcoreMesh(**_SC_MESH),
        scratch_types=[
            pltpu.VMEM((EPW + 16,), jnp.int32),
            pltpu.VMEM((EPW + 16,), jnp.int32),
            pltpu.VMEM((NWORK * RCH,), jnp.int32),
            pltpu.VMEM((NWORK + 16,), jnp.int32),
            pltpu.VMEM((NWORK + 16,), jnp.int32),
            pltpu.VMEM((16,), jnp.int32),
            pltpu.VMEM((N_NODE + 32,), jnp.float32),
        ],
    )
    def k(src_h, dst_h, slots_h, ncnt_h, cnt_h, sbuf, dbuf, ckbuf, pend, nch,
          stage, cbuf):
        c = lax.axis_index("c")
        w = lax.axis_index("s")
        wid = w * NCORE + c
        base_e = wid * EPW
        pltpu.sync_copy(src_h.at[pl.ds(base_e, EPW)], sbuf.at[pl.ds(0, EPW)])
        pltpu.sync_copy(dst_h.at[pl.ds(base_e, EPW)], dbuf.at[pl.ds(0, EPW)])
        iota = _iota16()
        zero = jnp.zeros((16,), jnp.int32)
        fz = jnp.zeros((16,), jnp.float32)
        for gg in range(3):
            pend[pl.ds(gg * 16, 16)] = zero
            nch[pl.ds(gg * 16, 16)] = zero

        def zc(r, carry):
            cbuf[pl.ds(r * 16, 16)] = fz
            return carry

        lax.fori_loop(0, (N_NODE + 32) // 16, zc, 0, unroll=False)

        def do_group(dstv, srcv, valid):
            ov = (dstv * 6554) >> 21
            locv = dstv - ov * OWN
            ov = jnp.where(valid, ov, 0)
            locv = jnp.where(valid, locv, TRASH)
            srcv = jnp.where(valid, srcv, 0)
            packv = srcv * 512 + locv
            dcnt = jnp.where(valid, dstv, N_NODE + 8)
            for l in range(16):
                o = ov[l]
                pk = packv[l]
                dd = dcnt[l]
                cv = cbuf[pl.ds(dd, 16)]
                cbuf[pl.ds(dd, 16)] = jnp.where(iota == 0, cv + 1.0, cv)
                pv = pend[pl.ds(o, 16)]
                p = pv[0]
                cbase = o * RCH + (p & ~jnp.int32(15))
                lane = p & 15
                v = ckbuf[pl.ds(cbase, 16)]
                ckbuf[pl.ds(cbase, 16)] = jnp.where(iota == lane, pk, v)
                p1 = p + 1

                @pl.when(p1 == RCH)
                def _flush():
                    nv = nch[pl.ds(o, 16)]
                    nc = nv[0]
                    off = ((wid * NWORK + o) * CAP + nc) * RCH
                    pltpu.sync_copy(ckbuf.at[pl.ds(o * RCH, RCH)],
                                    slots_h.at[pl.ds(off, RCH)])
                    nch[pl.ds(o, 16)] = jnp.where(iota == 0, nc + 1, nv)

                pend[pl.ds(o, 16)] = jnp.where(
                    iota == 0, jnp.where(p1 == RCH, 0, p1), pv)

        def step(g, carry):
            dstv = dbuf[pl.ds(g * 16, 16)]
            srcv = sbuf[pl.ds(g * 16, 16)]
            do_group(dstv, srcv, iota >= 0)
            return carry

        lax.fori_loop(0, NGRP, step, 0, unroll=False)
        dstv = dbuf[pl.ds(NGRP * 16, 16)]
        srcv = sbuf[pl.ds(NGRP * 16, 16)]
        do_group(dstv, srcv, iota < (EPW - NGRP * 16))

        for o in range(NWORK):
            pv = pend[pl.ds(o, 16)]
            p = pv[0]
            for gg in range(RCH // 16):
                gl = iota + gg * 16
                v = ckbuf[pl.ds(o * RCH + gg * 16, 16)]
                ckbuf[pl.ds(o * RCH + gg * 16, 16)] = jnp.where(
                    gl >= p, jnp.int32(TRASH), v)

            @pl.when(p > 0)
            def _flush():
                nv = nch[pl.ds(o, 16)]
                nc = nv[0]
                off = ((wid * NWORK + o) * CAP + nc) * RCH
                pltpu.sync_copy(ckbuf.at[pl.ds(o * RCH, RCH)],
                                slots_h.at[pl.ds(off, RCH)])
                nch[pl.ds(o, 16)] = jnp.where(iota == 0, nc + 1, nv)

        for gg in range(2):
            acc = jnp.zeros((16,), jnp.int32)
            for l in range(16):
                nv = nch[pl.ds(gg * 16 + l, 16)]
                acc = jnp.where(iota == l, nv[0], acc)
            stage[pl.ds(0, 16)] = acc
            pltpu.sync_copy(stage, ncnt_h.at[pl.ds(wid * NWORK + gg * 16, 16)])

        pltpu.sync_copy(cbuf.at[pl.ds(0, N_NODE)],
                        cnt_h.at[pl.ds(wid * N_NODE, N_NODE)])

    return k(src, dst)


def _segsum_sc(table, slots, ncnt):
    @functools.partial(
        pl.kernel,
        out_type=jax.ShapeDtypeStruct((N_NODE, H), jnp.float32),
        mesh=plsc.VectorSubcoreMesh(**_SC_MESH),
        scratch_types=[
            pltpu.VMEM((TAB_ROWS, H), jnp.float32),
            pltpu.VMEM((RCH, H), jnp.float32),
            pltpu.VMEM((RCH, H), jnp.float32),
            pltpu.VMEM((RCH + 16,), jnp.int32),
            pltpu.VMEM((RCH + 16,), jnp.int32),
            pltpu.VMEM((RCH,), jnp.int32),
            pltpu.VMEM((RCH,), jnp.int32),
            pltpu.VMEM((CADDR_MAX + 16,), jnp.int32),
            pltpu.VMEM((NWORK * NWORK + 16,), jnp.int32),
            pltpu.SemaphoreType.DMA,
            pltpu.SemaphoreType.DMA,
            pltpu.SemaphoreType.DMA,
            pltpu.SemaphoreType.DMA,
        ],
    )
    def k(table_h, slots_h, ncnt_h, agg_h,
          tab, rows_x, rows_y, pb_x, pb_y, sx, sy, caddr, ncb,
          lsem_x, lsem_y, gsem_x, gsem_y):
        c = lax.axis_index("c")
        w = lax.axis_index("s")
        o = w * NCORE + c
        iota = _iota16()
        fz = jnp.zeros((16,), jnp.float32)

        def zrow(r, carry):
            for jj in range(H // 16):
                tab[r, pl.ds(jj * 16, 16)] = fz
            return carry

        lax.fori_loop(0, TAB_ROWS, zrow, 0, unroll=False)

        pltpu.sync_copy(ncnt_h, ncb.at[pl.ds(0, NWORK * NWORK)])

        def build_w(wsc, t0):
            nv = ncb[pl.ds(wsc * NWORK + o, 16)]
            n_w = nv[0]

            def app(cc, t):
                addr = ((wsc * NWORK + o) * CAP + cc) * RCH
                b = t & ~jnp.int32(15)
                v = caddr[pl.ds(b, 16)]
                caddr[pl.ds(b, 16)] = jnp.where(iota == (t & 15), addr, v)
                return t + 1

            return lax.fori_loop(0, n_w, app, t0, unroll=False)

        t_total = jnp.int32(0)
        for wsc in range(NWORK):
            t_total = build_w(wsc, t_total)

        def accum(rows, pbuf):
            def arow(r2, carry):
                r0 = r2 * 2
                pv0 = pbuf[pl.ds(r0, 16)]
                pv1 = pbuf[pl.ds(r0 + 1, 16)]
                loc0 = pv0[0] & 511
                loc1 = pv1[0] & 511
                vals0 = [rows[r0, pl.ds(jj * 16, 16)] for jj in range(H // 16)]
                vals1 = [rows[r0 + 1, pl.ds(jj * 16, 16)] for jj in range(H // 16)]
                for jj in range(H // 16):
                    plsc.addupdate(tab.at[loc0, pl.ds(jj * 16, 16)], vals0[jj])
                for jj in range(H // 16):
                    plsc.addupdate(tab.at[loc1, pl.ds(jj * 16, 16)], vals1[jj])
                return carry

            lax.fori_loop(0, RCH // 2, arow, 0, unroll=False)

        def unpack(pbuf, sref):
            for gg in range(RCH // 16):
                pv = pbuf[pl.ds(gg * 16, 16)]
                sref[pl.ds(gg * 16, 16)] = lax.shift_right_logical(pv, 9)

        def chunk_addr(j):
            av = caddr[pl.ds(j, 16)]
            return pl.multiple_of(av[0], RCH)

        def stage(j, pbufA, sA, rowsA, lsemA, gsemA, pbufB, sB, rowsB, lsemB, gsemB):
            pltpu.make_async_copy(slots_h.at[pl.ds(0, RCH)],
                                  pbufA.at[pl.ds(0, RCH)], lsemA).wait()
            unpack(pbufA, sA)
            pltpu.async_copy(table_h.at[sA], rowsA, gsemA)

            @pl.when(j > 0)
            def _acc_prev():
                pltpu.make_async_copy(table_h.at[sB], rowsB, gsemB).wait()
                accum(rowsB, pbufB)

            @pl.when(j + 1 < t_total)
            def _next_load():
                pltpu.async_copy(slots_h.at[pl.ds(chunk_addr(j + 1), RCH)],
                                 pbufB.at[pl.ds(0, RCH)], lsemB)

        @pl.when(t_total > 0)
        def _prologue():
            pltpu.async_copy(slots_h.at[pl.ds(chunk_addr(jnp.int32(0)), RCH)],
                             pb_x.at[pl.ds(0, RCH)], lsem_x)

        def pipe(j, carry):
            @pl.when((j & 1) == 0)
            def _x():
                stage(j, pb_x, sx, rows_x, lsem_x, gsem_x,
                      pb_y, sy, rows_y, lsem_y, gsem_y)

            @pl.when((j & 1) == 1)
            def _y():
                stage(j, pb_y, sy, rows_y, lsem_y, gsem_y,
                      pb_x, sx, rows_x, lsem_x, gsem_x)

            return carry

        lax.fori_loop(0, t_total, pipe, 0, unroll=False)

        @pl.when((t_total > 0) & ((t_total & 1) == 1))
        def _epi_x():
            pltpu.make_async_copy(table_h.at[sx], rows_x, gsem_x).wait()
            accum(rows_x, pb_x)

        @pl.when((t_total > 0) & ((t_total & 1) == 0))
        def _epi_y():
            pltpu.make_async_copy(table_h.at[sy], rows_y, gsem_y).wait()
            accum(rows_y, pb_y)

        @pl.when(o < NWORK - 1)
        def _wb():
            pltpu.sync_copy(tab.at[pl.ds(0, OWN)], agg_h.at[pl.ds(o * OWN, OWN)])

        @pl.when(o == NWORK - 1)
        def _wb_last():
            nlast = N_NODE - (NWORK - 1) * OWN
            pltpu.sync_copy(tab.at[pl.ds(0, nlast)],
                            agg_h.at[pl.ds((NWORK - 1) * OWN, nlast)])

    return k(table, slots, ncnt)


def _gather2_sc(t0, t1, i0, i1):
    @functools.partial(
        pl.kernel,
        out_type=(jax.ShapeDtypeStruct((EL_EDGE, H), jnp.float32),
                  jax.ShapeDtypeStruct((EL_EDGE, H), jnp.float32)),
        mesh=plsc.VectorSubcoreMesh(**_SC_MESH),
        scratch_types=[
            pltpu.VMEM((GAT_CHUNK,), jnp.int32),
            pltpu.VMEM((GAT_CHUNK,), jnp.int32),
            pltpu.VMEM((GAT_CHUNK, H), jnp.float32),
            pltpu.VMEM((GAT_CHUNK, H), jnp.float32),
            pltpu.SemaphoreType.DMA,
            pltpu.SemaphoreType.DMA,
        ],
    )
    def k(t0_h, t1_h, i0_h, i1_h, g0_h, g1_h, i0_v, i1_v, r0_v, r1_v, s0, s1):
        c = lax.axis_index("c")
        w = lax.axis_index("s")
        wid = w * NCORE + c
        base0 = wid * GAT_PER_W

        def step(i, carry):
            base = base0 + i * GAT_CHUNK
            pltpu.sync_copy(i0_h.at[pl.ds(base, GAT_CHUNK)], i0_v)
            pltpu.sync_copy(i1_h.at[pl.ds(base, GAT_CHUNK)], i1_v)
            cp0 = pltpu.async_copy(t0_h.at[i0_v], r0_v, s0)
            cp1 = pltpu.async_copy(t1_h.at[i1_v], r1_v, s1)
            cp0.wait()
            cp1.wait()
            pltpu.sync_copy(r0_v, g0_h.at[pl.ds(base, GAT_CHUNK)])
            pltpu.sync_copy(r1_v, g1_h.at[pl.ds(base, GAT_CHUNK)])
            return carry

        lax.fori_loop(0, GAT_ITERS, step, 0)

    return k(t0, t1, i0, i1)


def _proj_tc(x, W, b, bm=1000):
    M, K = x.shape
    N = W.shape[1]

    def body(x_ref, w_ref, b_ref, o_ref):
        acc = jnp.dot(x_ref[...], w_ref[...], preferred_element_type=jnp.float32)
        o_ref[...] = _gelu(acc + b_ref[...])

    return pl.pallas_call(
        body,
        grid=(M // bm,),
        in_specs=[pl.BlockSpec((bm, K), lambda i: (i, 0)),
                  pl.BlockSpec((K, N), lambda i: (0, 0)),
                  pl.BlockSpec((1, N), lambda i: (0, 0))],
        out_specs=pl.BlockSpec((bm, N), lambda i: (i, 0)),
        out_shape=jax.ShapeDtypeStruct((M, N), jnp.float32),
    )(x, W, b.reshape(1, N))


def _combine_tc(agg, cnt_part, z_dst, Wl, Wr, b, bm=1000):
    M = agg.shape[0]

    def body(a_ref, c_ref, z_ref, wl_ref, wr_ref, b_ref, o_ref):
        cntc = jnp.sum(c_ref[...], axis=1)[:, None]
        mean = a_ref[...] / jnp.maximum(cntc, 1.0)
        acc = jnp.dot(mean, wl_ref[...], preferred_element_type=jnp.float32)
        acc += jnp.dot(z_ref[...], wr_ref[...], preferred_element_type=jnp.float32)
        o_ref[...] = _gelu(acc + b_ref[...])

    return pl.pallas_call(
        body,
        grid=(M // bm,),
        in_specs=[pl.BlockSpec((bm, H), lambda i: (i, 0)),
                  pl.BlockSpec((bm, NWORK), lambda i: (i, 0)),
                  pl.BlockSpec((bm, H), lambda i: (i, 0)),
                  pl.BlockSpec((H, H), lambda i: (0, 0)),
                  pl.BlockSpec((H, H), lambda i: (0, 0)),
                  pl.BlockSpec((1, H), lambda i: (0, 0))],
        out_specs=pl.BlockSpec((bm, H), lambda i: (i, 0)),
        out_shape=jax.ShapeDtypeStruct((M, H), jnp.float32),
    )(agg, cnt_part, z_dst, Wl, Wr, b.reshape(1, H))


def _edge_head_tc(zt_g, zs_g, Wa, Wb, Wc, b1, W2, b2, bm=1000):
    M = zt_g.shape[0]
    NO = W2.shape[1]

    def body(zt_ref, zs_ref, wa_ref, wb_ref, wc_ref, b1_ref, w2_ref, b2_ref, o_ref):
        zt = zt_ref[...]
        zs = zs_ref[...]
        d = jnp.abs(zt - zs)
        acc = jnp.dot(zt, wa_ref[...], preferred_element_type=jnp.float32)
        acc += jnp.dot(zs, wb_ref[...], preferred_element_type=jnp.float32)
        acc += jnp.dot(d, wc_ref[...], preferred_element_type=jnp.float32)
        g = _gelu(acc + b1_ref[...])
        o_ref[...] = jnp.dot(g, w2_ref[...], preferred_element_type=jnp.float32) + b2_ref[...]

    return pl.pallas_call(
        body,
        grid=(M // bm,),
        in_specs=[pl.BlockSpec((bm, H), lambda i: (i, 0)),
                  pl.BlockSpec((bm, H), lambda i: (i, 0)),
                  pl.BlockSpec((H, H), lambda i: (0, 0)),
                  pl.BlockSpec((H, H), lambda i: (0, 0)),
                  pl.BlockSpec((H, H), lambda i: (0, 0)),
                  pl.BlockSpec((1, H), lambda i: (0, 0)),
                  pl.BlockSpec((H, NO), lambda i: (0, 0)),
                  pl.BlockSpec((1, NO), lambda i: (0, 0))],
        out_specs=pl.BlockSpec((bm, NO), lambda i: (i, 0)),
        out_shape=jax.ShapeDtypeStruct((M, NO), jnp.float32),
    )(zt_g, zs_g, Wa, Wb, Wc, b1.reshape(1, H), W2, b2.reshape(1, NO))


def _node_heads_tc(z, W1, b1, W2, b2, bm=1000):
    M = z.shape[0]

    def body(z_ref, w1_ref, b1_ref, w2_ref, b2_ref, o_ref):
        h = _gelu(jnp.dot(z_ref[...], w1_ref[...], preferred_element_type=jnp.float32)
                  + b1_ref[...])
        o = jnp.dot(h, w2_ref[...], preferred_element_type=jnp.float32) + b2_ref[...]
        col = lax.broadcasted_iota(jnp.int32, o.shape, 1)
        o_ref[...] = jnp.where(col == 1, jax.nn.sigmoid(o), o)

    return pl.pallas_call(
        body,
        grid=(M // bm,),
        in_specs=[pl.BlockSpec((bm, H), lambda i: (i, 0)),
                  pl.BlockSpec((H, H), lambda i: (0, 0)),
                  pl.BlockSpec((1, H), lambda i: (0, 0)),
                  pl.BlockSpec((H, 2), lambda i: (0, 0)),
                  pl.BlockSpec((1, 2), lambda i: (0, 0))],
        out_specs=pl.BlockSpec((bm, 2), lambda i: (i, 0)),
        out_shape=jax.ShapeDtypeStruct((M, 2), jnp.float32),
    )(z, W1, b1.reshape(1, H), W2, b2.reshape(1, 2))


def kernel(x_transcript, x_symptom, edge_index_ts, edge_index_st, edge_label_index,
           Wt_proj, bt_proj, Ws_proj, bs_proj,
           Wl0_ts, bl0_ts, Wr0_ts, Wl0_st, bl0_st, Wr0_st,
           Wl1_ts, bl1_ts, Wr1_ts, Wl1_st, bl1_st, Wr1_st,
           We1, be1, We2, be2, Wb1, bb1, Wb2, bb2, Wsc1, bsc1, Wsc2, bsc2):
    ei_ts = edge_index_ts.astype(jnp.int32)
    ei_st = edge_index_st.astype(jnp.int32)
    eli = edge_label_index.astype(jnp.int32)

    slots_ts, ncnt_ts, cntp_ts = _route_sc(ei_ts[0], ei_ts[1])
    slots_st, ncnt_st, cntp_st = _route_sc(ei_st[0], ei_st[1])
    cntp_ts = cntp_ts.reshape(NWORK, N_NODE).T
    cntp_st = cntp_st.reshape(NWORK, N_NODE).T

    z_t = _proj_tc(x_transcript, Wt_proj, bt_proj)
    z_s = _proj_tc(x_symptom, Ws_proj, bs_proj)

    layers = ((Wl0_ts, bl0_ts, Wr0_ts, Wl0_st, bl0_st, Wr0_st),
              (Wl1_ts, bl1_ts, Wr1_ts, Wl1_st, bl1_st, Wr1_st))
    for (Wl_ts, bl_ts, Wr_ts, Wl_st, bl_st, Wr_st) in layers:
        agg_s = _segsum_sc(z_t, slots_ts, ncnt_ts)
        agg_t = _segsum_sc(z_s, slots_st, ncnt_st)
        new_s = _combine_tc(agg_s, cntp_ts, z_s, Wl_ts, Wr_ts, bl_ts)
        new_t = _combine_tc(agg_t, cntp_st, z_t, Wl_st, Wr_st, bl_st)
        z_t, z_s = new_t, new_s

    zt_g, zs_g = _gather2_sc(z_t, z_s, eli[0], eli[1])

    Wa, Wb, Wc = We1[:H], We1[H:2 * H], We1[2 * H:]
    edge_logits = _edge_head_tc(zt_g, zs_g, Wa, Wb, Wc, be1, We2, be2)

    W1 = jnp.concatenate([Wb1, Wsc1], axis=1)
    b1 = jnp.concatenate([bb1, bsc1])
    W2 = jnp.zeros((H, 2), jnp.float32)
    W2 = W2.at[:H // 2, 0].set(Wb2[:, 0]).at[H // 2:, 1].set(Wsc2[:, 0])
    b2 = jnp.stack([bb2[0], bsc2[0]])
    nh = _node_heads_tc(z_t, W1, b1, W2, b2)
    binary_logit = nh[:, 0]
    score_frac = nh[:, 1]

    return (edge_logits, binary_logit, score_frac, z_t, z_s)
